# baseline (speedup 1.0000x reference)
import functools
import jax
import jax.numpy as jnp
from jax import lax
from jax.experimental import pallas as pl
from jax.experimental.pallas import tpu as pltpu
from jax.experimental.pallas import tpu_sc as plsc

_BS = 64
_T = 2048
_D = 256
_NC = 2
_NS = 16
_NW = _NC * _NS
_BPW = _BS // _NW
_ROWS = _T // _NS
_CH = 128
_NCH = _T // _CH


def _sc_body(table_hbm, out_hbm, spmem, bufs, sem_a, sem_r, sem_w):
    cid = lax.axis_index("c")
    sid = lax.axis_index("s")

    r0 = sid * _ROWS
    pltpu.sync_copy(table_hbm.at[pl.ds(r0, _ROWS)], spmem.at[pl.ds(r0, _ROWS)])
    plsc.subcore_barrier()

    base = (sid * _NC + cid) * _BPW

    copy_a = pltpu.async_copy(spmem, out_hbm.at[base], sem_a)

    reads = [
        pltpu.make_async_copy(
            table_hbm.at[pl.ds(i * _CH, _CH)], bufs.at[i % 2], sem_r
        )
        for i in range(_NCH)
    ]
    writes = [
        pltpu.make_async_copy(
            bufs.at[i % 2], out_hbm.at[base + 1, pl.ds(i * _CH, _CH)], sem_w
        )
        for i in range(_NCH)
    ]
    reads[0].start()
    for i in range(_NCH):
        reads[i].wait()
        writes[i].start()
        if i + 1 < _NCH:
            if i >= 1:
                writes[i - 1].wait()
            reads[i + 1].start()
    writes[_NCH - 1].wait()

    copy_a.wait()


def kernel(mask, embed_weight):
    bs, t = mask.shape
    n_embed, d = embed_weight.shape

    mesh = plsc.VectorSubcoreMesh(core_axis_name="c", subcore_axis_name="s")
    k = functools.partial(
        pl.kernel,
        mesh=mesh,
        out_type=jax.ShapeDtypeStruct((bs, t, d), embed_weight.dtype),
        scratch_types=[
            pltpu.VMEM_SHARED((t, d), embed_weight.dtype),
            pltpu.VMEM((2, _CH, d), embed_weight.dtype),
            pltpu.SemaphoreType.DMA,
            pltpu.SemaphoreType.DMA,
            pltpu.SemaphoreType.DMA,
        ],
    )(_sc_body)
    return k(embed_weight[:t])


# speedup vs baseline: 2.8669x; 2.8669x over previous
import jax
import jax.numpy as jnp
from jax.experimental import pallas as pl
from jax.experimental.pallas import tpu as pltpu


def _make_body(bs):
    def body(emb_ref, out_ref, sem):
        copies = [
            pltpu.make_async_copy(emb_ref, out_ref.at[b], sem)
            for b in range(bs)
        ]
        for c in copies:
            c.start()
        for c in copies:
            c.wait()

    return body


def kernel(mask, embed_weight):
    bs, t = mask.shape
    n_embed, d = embed_weight.shape

    out = pl.pallas_call(
        _make_body(bs),
        in_specs=[pl.BlockSpec(memory_space=pltpu.MemorySpace.VMEM)],
        out_specs=pl.BlockSpec(memory_space=pl.ANY),
        out_shape=jax.ShapeDtypeStruct((bs, t, d), embed_weight.dtype),
        scratch_shapes=[pltpu.SemaphoreType.DMA],
    )(embed_weight[:t])
    return out


# speedup vs baseline: 2.8973x; 1.0106x over previous
import jax
import jax.numpy as jnp
from jax.experimental import pallas as pl
from jax.experimental.pallas import tpu as pltpu


def _make_body(bs, t):
    h = t // 2
    spans = [(0, h), (h, t - h)]

    def body(emb_ref, out_ref, vmem, lsem, wsem):
        loads = [
            pltpu.make_async_copy(
                emb_ref.at[pl.ds(o, n)], vmem.at[pl.ds(o, n)], lsem
            )
            for (o, n) in spans
        ]
        for l in loads:
            l.start()
        writes = []
        for i, (o, n) in enumerate(spans):
            loads[i].wait()
            half = [
                pltpu.make_async_copy(
                    vmem.at[pl.ds(o, n)],
                    out_ref.at[b, pl.ds(o, n)],
                    wsem,
                )
                for b in range(bs)
            ]
            for c in half:
                c.start()
            writes.extend(half)
        for c in writes:
            c.wait()

    return body


def kernel(mask, embed_weight):
    bs, t = mask.shape
    n_embed, d = embed_weight.shape
    emb = embed_weight[:t]

    out = pl.pallas_call(
        _make_body(bs, t),
        in_specs=[pl.BlockSpec(memory_space=pl.ANY)],
        out_specs=pl.BlockSpec(memory_space=pl.ANY),
        out_shape=jax.ShapeDtypeStruct((bs, t, d), embed_weight.dtype),
        scratch_shapes=[
            pltpu.VMEM((t, d), embed_weight.dtype),
            pltpu.SemaphoreType.DMA,
            pltpu.SemaphoreType.DMA,
        ],
    )(emb)
    return out
